# baseline (speedup 1.0000x reference)
import functools

import jax
import jax.numpy as jnp
from jax.experimental import pallas as pl
from jax.experimental.pallas import tpu as pltpu


def _eye(n, dtype):
    r = jax.lax.broadcasted_iota(jnp.int32, (n, n), 0)
    c = jax.lax.broadcasted_iota(jnp.int32, (n, n), 1)
    return (r == c).astype(dtype)


def _gcn_body(dlf_ref, qm_ref, spk_ref, w1_ref, b1_ref, cw_ref, cb_ref,
              l_hbm, a_hbm, v_hbm, out_ref,
              lb, ab, vb, stage, lsem, ssem, *, n_dia, dia, num_k, nch):
    total = n_dia * dia
    d = lb.shape[1]
    f32 = jnp.float32
    rows = total // nch
    grp = n_dia // nch

    def dot(x, y):
        return jnp.dot(x, y, preferred_element_type=f32)

    def pmul(p1, p2):
        m1, n1 = p1
        m2, n2 = p2
        return (dot(m1, m2), dot(m1, n2) + dot(n1, m2) + 3.0 * dot(n1, n2))

    for c in range(nch):
        sl = pl.ds(c * rows, rows)
        pltpu.make_async_copy(l_hbm.at[sl, :], lb.at[sl, :], lsem.at[0, c]).start()
        pltpu.make_async_copy(a_hbm.at[sl, :], ab.at[sl, :], lsem.at[1, c]).start()
        pltpu.make_async_copy(v_hbm.at[sl, :], vb.at[sl, :], lsem.at[2, c]).start()

    scale = jnp.sum(dlf_ref[0, :]) / f32(dlf_ref.shape[1] * dia)
    spk_add = dot(qm_ref[...], spk_ref[...])
    w1t = w1_ref[...].T
    b1 = b1_ref[...]

    ident = _eye(d, f32)
    inv = f32(1.0 / (dia + 2))
    fdia = f32(dia)

    w0 = cw_ref[0].T * inv
    b0 = cb_ref[0:1, :]
    P = (ident - w0, w0)
    Q = (w0, jnp.zeros_like(w0))
    R = (ident + (fdia - 1.0) * w0, w0)
    rho = b0
    sig = fdia * b0
    for k in range(1, num_k):
        wk = cw_ref[k].T * inv
        bk = cb_ref[k:k + 1, :]
        A = (ident - wk, wk)
        P = pmul(P, A)
        Q = tuple(x + y for x, y in zip(pmul(Q, A),
                                        (dot(R[0], wk), dot(R[1], wk))))
        new_rho = dot(rho, ident + 2.0 * wk) + dot(sig, wk) + bk
        sig = dot(sig, ident + (fdia + 2.0) * wk) + fdia * bk
        rho = new_rho
        if k + 1 < num_k:
            R = pmul(R, (ident + (fdia - 1.0) * wk, wk))
    MP, NP = P
    MQ, NQ = Q

    for c in range(nch):
        sl = pl.ds(c * rows, rows)
        pltpu.make_async_copy(l_hbm.at[sl, :], lb.at[sl, :], lsem.at[0, c]).wait()
        pltpu.make_async_copy(a_hbm.at[sl, :], ab.at[sl, :], lsem.at[1, c]).wait()
        pltpu.make_async_copy(v_hbm.at[sl, :], vb.at[sl, :], lsem.at[2, c]).wait()

        xl = (lb[sl, :] + spk_add[c * rows:(c + 1) * rows, :]) * scale
        xa = ab[sl, :] * scale
        xv = vb[sl, :] * scale
        hl = dot(xl, w1t) + b1
        ha = dot(xa, w1t) + b1
        hv = dot(xv, w1t) + b1

        rowtot = hl + ha + hv
        ul = jnp.sum(hl.reshape(grp, dia, d), axis=1)
        ua = jnp.sum(ha.reshape(grp, dia, d), axis=1)
        uv = jnp.sum(hv.reshape(grp, dia, d), axis=1)
        utot = ul + ua + uv
        rt_np = dot(rowtot, NP)
        ut_nq = dot(utot, NQ) + rho

        gs = []
        for hm, um in ((hl, ul), (ha, ua), (hv, uv)):
            dm = dot(um, MQ) + ut_nq
            db = jnp.broadcast_to(dm.reshape(grp, 1, d),
                                  (grp, dia, d)).reshape(rows, d)
            gs.append(dot(hm, MP) + rt_np + db)

        for col, val in enumerate((xl, hl, gs[0], xa, ha, gs[1],
                                   xv, hv, gs[2])):
            stage[sl, col * d:(col + 1) * d] = val
        pltpu.make_async_copy(stage.at[sl, :], out_ref.at[sl, :],
                              ssem.at[c]).start()

    for c in range(nch):
        sl = pl.ds(c * rows, rows)
        pltpu.make_async_copy(stage.at[sl, :], out_ref.at[sl, :],
                              ssem.at[c]).wait()


def kernel(a, v, l, qmask, dia_len, epoch, spk_emb, fc1_w, fc1_b, conv_w,
           conv_b):
    del epoch
    total, d = a.shape
    n_dia = dia_len.shape[0]
    dia = qmask.shape[0]
    num_k = conv_w.shape[0]
    nspk = qmask.shape[2]

    nch = 2
    while n_dia % nch:
        nch -= 1

    qm = jnp.transpose(qmask, (1, 0, 2)).reshape(total, nspk)
    dlf = dia_len.astype(jnp.float32).reshape(1, n_dia)
    b1 = fc1_b.reshape(1, -1)

    body = functools.partial(_gcn_body, n_dia=n_dia, dia=dia, num_k=num_k,
                             nch=nch)
    hbm = pl.BlockSpec(memory_space=pltpu.MemorySpace.HBM)
    out = pl.pallas_call(
        body,
        in_specs=[
            pl.BlockSpec((1, n_dia), lambda: (0, 0)),
            pl.BlockSpec((total, nspk), lambda: (0, 0)),
            pl.BlockSpec((spk_emb.shape[0], d), lambda: (0, 0)),
            pl.BlockSpec((d, d), lambda: (0, 0)),
            pl.BlockSpec((1, d), lambda: (0, 0)),
            pl.BlockSpec((num_k, d, d), lambda: (0, 0, 0)),
            pl.BlockSpec((num_k, d), lambda: (0, 0)),
            hbm,
            hbm,
            hbm,
        ],
        out_specs=hbm,
        out_shape=jax.ShapeDtypeStruct((total, 9 * d), jnp.float32),
        scratch_shapes=[
            pltpu.VMEM((total, d), jnp.float32),
            pltpu.VMEM((total, d), jnp.float32),
            pltpu.VMEM((total, d), jnp.float32),
            pltpu.VMEM((total, 9 * d), jnp.float32),
            pltpu.SemaphoreType.DMA((3, nch)),
            pltpu.SemaphoreType.DMA((nch,)),
        ],
    )(dlf, qm, spk_emb, fc1_w, b1, conv_w, conv_b, l, a, v)
    return out


# speedup vs baseline: 1.0651x; 1.0651x over previous
import functools

import jax
import jax.numpy as jnp
from jax.experimental import pallas as pl
from jax.experimental.pallas import tpu as pltpu


def _eye(n, dtype):
    r = jax.lax.broadcasted_iota(jnp.int32, (n, n), 0)
    c = jax.lax.broadcasted_iota(jnp.int32, (n, n), 1)
    return (r == c).astype(dtype)


def _gcn_body(dlf_ref, qm_ref, spk_ref, w1_ref, b1_ref, cw_ref, cb_ref,
              l_hbm, a_hbm, v_hbm, out_ref,
              lb, ab, vb, stage, lsem, ssem, *, n_dia, dia, num_k, nch):
    total = n_dia * dia
    d = lb.shape[1]
    f32 = jnp.float32
    rows = total // nch
    grp = n_dia // nch

    def dot(x, y):
        return jnp.dot(x, y, preferred_element_type=f32)

    def pmul(p1, p2):
        m1, n1 = p1
        m2, n2 = p2
        return (dot(m1, m2), dot(m1, n2) + dot(n1, m2) + 3.0 * dot(n1, n2))

    for c in range(nch):
        sl = pl.ds(c * rows, rows)
        pltpu.make_async_copy(l_hbm.at[sl, :], lb.at[sl, :], lsem.at[0, c]).start()
        pltpu.make_async_copy(a_hbm.at[sl, :], ab.at[sl, :], lsem.at[1, c]).start()
        pltpu.make_async_copy(v_hbm.at[sl, :], vb.at[sl, :], lsem.at[2, c]).start()

    scale = jnp.sum(dlf_ref[0, :]) / f32(dlf_ref.shape[1] * dia)
    spk_add = dot(qm_ref[...], spk_ref[...])
    w1t = w1_ref[...].T
    b1 = b1_ref[...]

    ident = _eye(d, f32)
    inv = f32(1.0 / (dia + 2))
    fdia = f32(dia)

    w0 = cw_ref[0].T * inv
    b0 = cb_ref[0:1, :]
    P = (ident - w0, w0)
    Q = (w0, jnp.zeros_like(w0))
    R = (ident + (fdia - 1.0) * w0, w0)
    rho = b0
    sig = fdia * b0
    for k in range(1, num_k):
        wk = cw_ref[k].T * inv
        bk = cb_ref[k:k + 1, :]
        A = (ident - wk, wk)
        P = pmul(P, A)
        Q = tuple(x + y for x, y in zip(pmul(Q, A),
                                        (dot(R[0], wk), dot(R[1], wk))))
        new_rho = dot(rho, ident + 2.0 * wk) + dot(sig, wk) + bk
        sig = dot(sig, ident + (fdia + 2.0) * wk) + fdia * bk
        rho = new_rho
        if k + 1 < num_k:
            R = pmul(R, (ident + (fdia - 1.0) * wk, wk))
    MP, NP = P
    MQ, NQ = Q

    ri = jax.lax.broadcasted_iota(jnp.int32, (grp, rows), 0)
    ci = jax.lax.broadcasted_iota(jnp.int32, (grp, rows), 1)
    seg = (ci // dia == ri).astype(f32)

    for c in range(nch):
        sl = pl.ds(c * rows, rows)
        pltpu.make_async_copy(l_hbm.at[sl, :], lb.at[sl, :], lsem.at[0, c]).wait()
        pltpu.make_async_copy(a_hbm.at[sl, :], ab.at[sl, :], lsem.at[1, c]).wait()
        pltpu.make_async_copy(v_hbm.at[sl, :], vb.at[sl, :], lsem.at[2, c]).wait()

        xl = (lb[sl, :] + spk_add[c * rows:(c + 1) * rows, :]) * scale
        xa = ab[sl, :] * scale
        xv = vb[sl, :] * scale
        hl = dot(xl, w1t) + b1
        ha = dot(xa, w1t) + b1
        hv = dot(xv, w1t) + b1

        rowtot = hl + ha + hv
        ul = dot(seg, hl)
        ua = dot(seg, ha)
        uv = dot(seg, hv)
        utot = ul + ua + uv
        rt_np = dot(rowtot, NP)
        ut_nq = dot(utot, NQ) + rho

        gs = []
        for hm, um in ((hl, ul), (ha, ua), (hv, uv)):
            dm = dot(um, MQ) + ut_nq
            db = jnp.broadcast_to(dm.reshape(grp, 1, d),
                                  (grp, dia, d)).reshape(rows, d)
            gs.append(dot(hm, MP) + rt_np + db)

        for col, val in enumerate((xl, hl, gs[0], xa, ha, gs[1],
                                   xv, hv, gs[2])):
            stage[sl, col * d:(col + 1) * d] = val
        pltpu.make_async_copy(stage.at[sl, :], out_ref.at[sl, :],
                              ssem.at[c]).start()

    for c in range(nch):
        sl = pl.ds(c * rows, rows)
        pltpu.make_async_copy(stage.at[sl, :], out_ref.at[sl, :],
                              ssem.at[c]).wait()


def kernel(a, v, l, qmask, dia_len, epoch, spk_emb, fc1_w, fc1_b, conv_w,
           conv_b):
    del epoch
    total, d = a.shape
    n_dia = dia_len.shape[0]
    dia = qmask.shape[0]
    num_k = conv_w.shape[0]
    nspk = qmask.shape[2]

    nch = 5
    while n_dia % nch:
        nch -= 1

    qm = jnp.transpose(qmask, (1, 0, 2)).reshape(total, nspk)
    dlf = dia_len.astype(jnp.float32).reshape(1, n_dia)
    b1 = fc1_b.reshape(1, -1)

    body = functools.partial(_gcn_body, n_dia=n_dia, dia=dia, num_k=num_k,
                             nch=nch)
    hbm = pl.BlockSpec(memory_space=pltpu.MemorySpace.HBM)
    out = pl.pallas_call(
        body,
        in_specs=[
            pl.BlockSpec((1, n_dia), lambda: (0, 0)),
            pl.BlockSpec((total, nspk), lambda: (0, 0)),
            pl.BlockSpec((spk_emb.shape[0], d), lambda: (0, 0)),
            pl.BlockSpec((d, d), lambda: (0, 0)),
            pl.BlockSpec((1, d), lambda: (0, 0)),
            pl.BlockSpec((num_k, d, d), lambda: (0, 0, 0)),
            pl.BlockSpec((num_k, d), lambda: (0, 0)),
            hbm,
            hbm,
            hbm,
        ],
        out_specs=hbm,
        out_shape=jax.ShapeDtypeStruct((total, 9 * d), jnp.float32),
        scratch_shapes=[
            pltpu.VMEM((total, d), jnp.float32),
            pltpu.VMEM((total, d), jnp.float32),
            pltpu.VMEM((total, d), jnp.float32),
            pltpu.VMEM((total, 9 * d), jnp.float32),
            pltpu.SemaphoreType.DMA((3, nch)),
            pltpu.SemaphoreType.DMA((nch,)),
        ],
    )(dlf, qm, spk_emb, fc1_w, b1, conv_w, conv_b, l, a, v)
    return out


# speedup vs baseline: 1.3397x; 1.2578x over previous
import functools

import jax
import jax.numpy as jnp
from jax.experimental import pallas as pl
from jax.experimental.pallas import tpu as pltpu


def _eye(n, dtype):
    r = jax.lax.broadcasted_iota(jnp.int32, (n, n), 0)
    c = jax.lax.broadcasted_iota(jnp.int32, (n, n), 1)
    return (r == c).astype(dtype)


def _gcn_body(dli_ref, qmr_ref, spk_ref, w1_ref, b1_ref, cw_ref, cb_ref,
              l_hbm, a_hbm, v_hbm, out_ref,
              lb, ab, vb, stage, lsem, ssem, *, n_dia, dia, num_k, nch):
    total = n_dia * dia
    d = lb.shape[1]
    f32 = jnp.float32
    rows = total // nch
    grp = n_dia // nch

    def dot(x, y):
        return jnp.dot(x, y, preferred_element_type=f32)

    def pmul(p1, p2):
        m1, n1 = p1
        m2, n2 = p2
        return (dot(m1, m2), dot(m1, n2) + dot(n1, m2) + 3.0 * dot(n1, n2))

    def iota2(shape, dim):
        return jax.lax.broadcasted_iota(jnp.int32, shape, dim)

    for c in range(nch):
        sl = pl.ds(c * rows, rows)
        pltpu.make_async_copy(l_hbm.at[sl, :], lb.at[sl, :], lsem.at[0, c]).start()
        pltpu.make_async_copy(a_hbm.at[sl, :], ab.at[sl, :], lsem.at[1, c]).start()
        pltpu.make_async_copy(v_hbm.at[sl, :], vb.at[sl, :], lsem.at[2, c]).start()

    scale = jnp.sum(dli_ref[0, :].astype(f32)) / f32(dli_ref.shape[1] * dia)
    sel = (iota2((qmr_ref.shape[1], n_dia), 0) ==
           2 * iota2((qmr_ref.shape[1], n_dia), 1) + 1).astype(f32)
    q1 = dot(qmr_ref[...], sel)
    p1 = (iota2((total, dia), 0) % dia == iota2((total, dia), 1)).astype(f32)
    p2 = (iota2((total, n_dia), 0) // dia ==
          iota2((total, n_dia), 1)).astype(f32)
    w50 = jnp.broadcast_to(spk_ref[1:2, :] - spk_ref[0:1, :], (n_dia, d))
    spk_add = dot(dot(p1, q1) * p2, w50) + spk_ref[0:1, :]
    w1t = w1_ref[...].T
    b1 = b1_ref[...]

    ident = _eye(d, f32)
    inv = f32(1.0 / (dia + 2))
    fdia = f32(dia)

    w0 = cw_ref[0].T * inv
    b0 = cb_ref[0:1, :]
    P = (ident - w0, w0)
    Q = (w0, jnp.zeros_like(w0))
    R = (ident + (fdia - 1.0) * w0, w0)
    rho = b0
    sig = fdia * b0
    for k in range(1, num_k):
        wk = cw_ref[k].T * inv
        bk = cb_ref[k:k + 1, :]
        A = (ident - wk, wk)
        P = pmul(P, A)
        Q = tuple(x + y for x, y in zip(pmul(Q, A),
                                        (dot(R[0], wk), dot(R[1], wk))))
        new_rho = dot(rho, ident + 2.0 * wk) + dot(sig, wk) + bk
        sig = dot(sig, ident + (fdia + 2.0) * wk) + fdia * bk
        rho = new_rho
        if k + 1 < num_k:
            R = pmul(R, (ident + (fdia - 1.0) * wk, wk))
    MP, NP = P
    MQ, NQ = Q

    ri = jax.lax.broadcasted_iota(jnp.int32, (grp, rows), 0)
    ci = jax.lax.broadcasted_iota(jnp.int32, (grp, rows), 1)
    seg = (ci // dia == ri).astype(f32)

    for c in range(nch):
        sl = pl.ds(c * rows, rows)
        pltpu.make_async_copy(l_hbm.at[sl, :], lb.at[sl, :], lsem.at[0, c]).wait()
        pltpu.make_async_copy(a_hbm.at[sl, :], ab.at[sl, :], lsem.at[1, c]).wait()
        pltpu.make_async_copy(v_hbm.at[sl, :], vb.at[sl, :], lsem.at[2, c]).wait()

        xl = (lb[sl, :] + spk_add[c * rows:(c + 1) * rows, :]) * scale
        xa = ab[sl, :] * scale
        xv = vb[sl, :] * scale
        hl = dot(xl, w1t) + b1
        ha = dot(xa, w1t) + b1
        hv = dot(xv, w1t) + b1

        rowtot = hl + ha + hv
        ul = dot(seg, hl)
        ua = dot(seg, ha)
        uv = dot(seg, hv)
        utot = ul + ua + uv
        rt_np = dot(rowtot, NP)
        ut_nq = dot(utot, NQ) + rho

        gs = []
        for hm, um in ((hl, ul), (ha, ua), (hv, uv)):
            dm = dot(um, MQ) + ut_nq
            db = jnp.broadcast_to(dm.reshape(grp, 1, d),
                                  (grp, dia, d)).reshape(rows, d)
            gs.append(dot(hm, MP) + rt_np + db)

        for col, val in enumerate((xl, hl, gs[0], xa, ha, gs[1],
                                   xv, hv, gs[2])):
            stage[sl, col * d:(col + 1) * d] = val
        pltpu.make_async_copy(stage.at[sl, :], out_ref.at[sl, :],
                              ssem.at[c]).start()

    for c in range(nch):
        sl = pl.ds(c * rows, rows)
        pltpu.make_async_copy(stage.at[sl, :], out_ref.at[sl, :],
                              ssem.at[c]).wait()


def kernel(a, v, l, qmask, dia_len, epoch, spk_emb, fc1_w, fc1_b, conv_w,
           conv_b):
    del epoch
    total, d = a.shape
    n_dia = dia_len.shape[0]
    dia = qmask.shape[0]
    num_k = conv_w.shape[0]
    nspk = qmask.shape[2]

    nch = 5
    while n_dia % nch:
        nch -= 1

    qmr = qmask.reshape(dia, n_dia * nspk)
    dli = dia_len.reshape(1, n_dia)
    b1 = fc1_b.reshape(1, -1)

    body = functools.partial(_gcn_body, n_dia=n_dia, dia=dia, num_k=num_k,
                             nch=nch)
    hbm = pl.BlockSpec(memory_space=pltpu.MemorySpace.HBM)
    out = pl.pallas_call(
        body,
        in_specs=[
            pl.BlockSpec((1, n_dia), lambda: (0, 0)),
            pl.BlockSpec((dia, n_dia * nspk), lambda: (0, 0)),
            pl.BlockSpec((spk_emb.shape[0], d), lambda: (0, 0)),
            pl.BlockSpec((d, d), lambda: (0, 0)),
            pl.BlockSpec((1, d), lambda: (0, 0)),
            pl.BlockSpec((num_k, d, d), lambda: (0, 0, 0)),
            pl.BlockSpec((num_k, d), lambda: (0, 0)),
            hbm,
            hbm,
            hbm,
        ],
        out_specs=hbm,
        out_shape=jax.ShapeDtypeStruct((total, 9 * d), jnp.float32),
        scratch_shapes=[
            pltpu.VMEM((total, d), jnp.float32),
            pltpu.VMEM((total, d), jnp.float32),
            pltpu.VMEM((total, d), jnp.float32),
            pltpu.VMEM((total, 9 * d), jnp.float32),
            pltpu.SemaphoreType.DMA((3, nch)),
            pltpu.SemaphoreType.DMA((nch,)),
        ],
    )(dli, qmr, spk_emb, fc1_w, b1, conv_w, conv_b, l, a, v)
    return out
